# baseline (speedup 1.0000x reference)
import functools

import jax
import jax.numpy as jnp
from jax.experimental import pallas as pl

Q = 8
K = 1024
D = 256
B, N = 8, 1024
M = 1024
TOKENS = B * N


def _rvq_kernel(x_ref, cb_ref, qout_ref, idx_ref, loss_ref):
    i = pl.program_id(0)

    @pl.when(i == 0)
    def _init():
        loss_ref[...] = jnp.zeros_like(loss_ref)

    r = x_ref[...]
    qacc = jnp.zeros_like(r)
    col = jax.lax.broadcasted_iota(jnp.int32, (M, K), 1)
    for q in range(Q):
        e = cb_ref[q]
        esum = jnp.sum(e * e, axis=1)
        rsum = jnp.sum(r * r, axis=1, keepdims=True)
        prod = jax.lax.dot_general(
            r.astype(jnp.bfloat16), e.astype(jnp.bfloat16),
            (((1,), (1,)), ((), ())),
            preferred_element_type=jnp.float32)
        dist = (rsum - 2.0 * prod) + esum[None, :]
        mind = jnp.min(dist, axis=1, keepdims=True)
        idx = jnp.min(jnp.where(dist == mind, col, K), axis=1)
        onehot = (col == idx[:, None]).astype(jnp.float32)
        quant = jax.lax.dot_general(
            onehot, e, (((1,), (0,)), ((), ())),
            preferred_element_type=jnp.float32,
            precision=jax.lax.Precision.HIGHEST)
        newr = r - quant
        loss_ref[q, :] += jnp.sum(newr * newr)
        qacc = qacc + quant
        r = newr
        idx_ref[q, :] = idx
    qout_ref[...] = qacc


@jax.jit
def kernel(x, codebooks):
    flat = x.reshape(TOKENS, D)
    grid = (TOKENS // M,)
    qout, idx_t, loss_acc = pl.pallas_call(
        _rvq_kernel,
        grid=grid,
        in_specs=[
            pl.BlockSpec((M, D), lambda i: (i, 0)),
            pl.BlockSpec((Q, K, D), lambda i: (0, 0, 0)),
        ],
        out_specs=[
            pl.BlockSpec((M, D), lambda i: (i, 0)),
            pl.BlockSpec((Q, M), lambda i: (0, i)),
            pl.BlockSpec((Q, 128), lambda i: (0, 0)),
        ],
        out_shape=[
            jax.ShapeDtypeStruct((TOKENS, D), jnp.float32),
            jax.ShapeDtypeStruct((Q, TOKENS), jnp.int32),
            jax.ShapeDtypeStruct((Q, 128), jnp.float32),
        ],
    )(flat, codebooks)
    quantized_out = qout.reshape(B, N, D)
    all_indices = idx_t.T.reshape(B, N, Q)
    all_losses = loss_acc[:, 0] / jnp.float32(TOKENS * D)
    return quantized_out, all_indices, all_losses


# speedup vs baseline: 1.8899x; 1.8899x over previous
import jax
import jax.numpy as jnp
from jax.experimental import pallas as pl

Q = 8
K = 1024
D = 256
B, N = 8, 1024
M = 1024
TOKENS = B * N


def _rvq_kernel(x_ref, cb_ref, hi_ref, mid_ref, lo_ref,
                qout_ref, idx_ref, loss_ref):
    i = pl.program_id(0)

    @pl.when(i == 0)
    def _init():
        loss_ref[...] = jnp.zeros_like(loss_ref)

    r = x_ref[...]
    qacc = jnp.zeros_like(r)
    col = jax.lax.broadcasted_iota(jnp.int32, (M, K), 1)

    def mm(a, b):
        return jax.lax.dot_general(
            a, b, (((1,), (0,)), ((), ())),
            preferred_element_type=jnp.float32)

    for q in range(Q):
        e = cb_ref[q]
        e_hi = hi_ref[q]
        esum = jnp.sum(e * e, axis=1)
        rsum = jnp.sum(r * r, axis=1, keepdims=True)
        prod = mm(r.astype(jnp.bfloat16), e_hi.T)
        dist = (rsum - 2.0 * prod) + esum[None, :]
        mind = jnp.min(dist, axis=1, keepdims=True)
        idx = jnp.min(jnp.where(dist == mind, col, K), axis=1)
        oh = (col == idx[:, None]).astype(jnp.bfloat16)
        quant = (mm(oh, e_hi) + mm(oh, mid_ref[q])) + mm(oh, lo_ref[q])
        newr = r - quant
        loss_ref[q, :] += jnp.sum(newr * newr)
        qacc = qacc + quant
        r = newr
        idx_ref[q, :] = idx
    qout_ref[...] = qacc


@jax.jit
def kernel(x, codebooks):
    flat = x.reshape(TOKENS, D)
    cb_hi = codebooks.astype(jnp.bfloat16)
    r1 = codebooks - cb_hi.astype(jnp.float32)
    cb_mid = r1.astype(jnp.bfloat16)
    cb_lo = (r1 - cb_mid.astype(jnp.float32)).astype(jnp.bfloat16)
    grid = (TOKENS // M,)
    qout, idx_t, loss_acc = pl.pallas_call(
        _rvq_kernel,
        grid=grid,
        in_specs=[
            pl.BlockSpec((M, D), lambda i: (i, 0)),
            pl.BlockSpec((Q, K, D), lambda i: (0, 0, 0)),
            pl.BlockSpec((Q, K, D), lambda i: (0, 0, 0)),
            pl.BlockSpec((Q, K, D), lambda i: (0, 0, 0)),
            pl.BlockSpec((Q, K, D), lambda i: (0, 0, 0)),
        ],
        out_specs=[
            pl.BlockSpec((M, D), lambda i: (i, 0)),
            pl.BlockSpec((Q, M), lambda i: (0, i)),
            pl.BlockSpec((Q, 128), lambda i: (0, 0)),
        ],
        out_shape=[
            jax.ShapeDtypeStruct((TOKENS, D), jnp.float32),
            jax.ShapeDtypeStruct((Q, TOKENS), jnp.int32),
            jax.ShapeDtypeStruct((Q, 128), jnp.float32),
        ],
    )(flat, codebooks, cb_hi, cb_mid, cb_lo)
    quantized_out = qout.reshape(B, N, D)
    all_indices = idx_t.T.reshape(B, N, Q)
    all_losses = loss_acc[:, 0] / jnp.float32(TOKENS * D)
    return quantized_out, all_indices, all_losses
